# Initial kernel scaffold; baseline (speedup 1.0000x reference)
import jax
import jax.numpy as jnp
from jax.experimental import pallas as pl


def kernel(xs, edge_indexs, W1, b1, W2, b2):
    raise NotImplementedError("write your pallas kernel here")


# speedup vs baseline: 16.3132x; 16.3132x over previous
import functools

import jax
import jax.numpy as jnp
from jax import lax
from jax.experimental import pallas as pl
from jax.experimental.pallas import tpu as pltpu
from jax.experimental.pallas import tpu_sc as plsc

N = 10000
E = 320000
F = 128
NC = 2
NS = 16
NW = NC * NS
K = 80
EPT = E // NW
NCHUNK = EPT // K
ZCH = 640
NPAD = NS * ZCH
TAIL = N - (NS - 1) * ZCH

_mesh = plsc.VectorSubcoreMesh(core_axis_name="c", subcore_axis_name="s")


@functools.partial(
    pl.kernel,
    out_type=jax.ShapeDtypeStruct((NC * 2, 1, NPAD), jnp.float32),
    mesh=_mesh,
    scratch_types=[
        pltpu.VMEM((NCHUNK, K), jnp.int32),
        pltpu.VMEM((K,), jnp.float32),
        pltpu.VMEM((ZCH,), jnp.float32),
        pltpu.VMEM_SHARED((NPAD,), jnp.float32),
        pltpu.VMEM_SHARED((NPAD,), jnp.float32),
    ],
)
def _hist_kernel(dsts, degp, dst_v, ones_v, zbuf, tb0, tb1):
    c = lax.axis_index("c")
    s = lax.axis_index("s")
    wid = c * NS + s
    for i in range(K // 16):
        ones_v[pl.ds(i * 16, 16)] = jnp.full((16,), 1.0, jnp.float32)
    for i in range(ZCH // 16):
        zbuf[pl.ds(i * 16, 16)] = jnp.zeros((16,), jnp.float32)
    tables = (tb0, tb1)
    for tb in tables:
        pltpu.sync_copy(zbuf, tb.at[pl.ds(s * ZCH, ZCH)])
    plsc.subcore_barrier()
    for b in range(2):
        tb = tables[b]
        pltpu.sync_copy(dsts.at[b, wid], dst_v)

        def body(j, _):
            pltpu.sync_copy(ones_v, tb.at[dst_v.at[j]], add=True)
            return ()

        lax.fori_loop(0, NCHUNK, body, ())
    plsc.subcore_barrier()
    for b in range(2):
        pltpu.sync_copy(tables[b].at[pl.ds(s * ZCH, ZCH)],
                        degp.at[c * 2 + b, 0, pl.ds(s * ZCH, ZCH)])


@functools.partial(
    pl.kernel,
    out_type=jax.ShapeDtypeStruct((NC, N, F), jnp.float32),
    mesh=_mesh,
    scratch_types=[
        pltpu.VMEM((NCHUNK, K), jnp.int32),
        pltpu.VMEM((NCHUNK, K), jnp.int32),
        pltpu.VMEM((K, F), jnp.float32),
        pltpu.VMEM_SHARED((N, F), jnp.float32),
    ],
)
def _agg_kernel(g, zer, srcs, dsts, p_out, src_v, dst_v, rbuf, accum):
    c = lax.axis_index("c")
    s = lax.axis_index("s")
    wid = c * NS + s

    def rows_sliced(fn):
        @pl.when(s < NS - 1)
        def _():
            fn(pl.ds(s * ZCH, ZCH))

        @pl.when(s == NS - 1)
        def _():
            fn(pl.ds((NS - 1) * ZCH, TAIL))

    def init(rows):
        @pl.when(c == 0)
        def _():
            pltpu.sync_copy(g.at[rows], accum.at[rows])

        @pl.when(c != 0)
        def _():
            pltpu.sync_copy(zer.at[rows], accum.at[rows])

    rows_sliced(init)
    pltpu.sync_copy(srcs.at[wid], src_v)
    pltpu.sync_copy(dsts.at[wid], dst_v)
    plsc.subcore_barrier()

    def body(j, _):
        pltpu.sync_copy(g.at[src_v.at[j]], rbuf)
        pltpu.sync_copy(rbuf, accum.at[dst_v.at[j]], add=True)
        return ()

    lax.fori_loop(0, NCHUNK, body, ())
    plsc.subcore_barrier()
    rows_sliced(lambda rows: pltpu.sync_copy(accum.at[rows], p_out.at[c, rows]))


def _dinv_body(degp_ref, o_ref):
    o_ref[...] = lax.rsqrt(degp_ref[0] + degp_ref[1] + 1.0)


def _dinv_call(degp):
    return pl.pallas_call(
        _dinv_body,
        out_shape=jax.ShapeDtypeStruct((2, NPAD), jnp.float32),
    )(degp)


_RB = 1000


def _m1_body(dinv_ref, x_ref, w_ref, o_ref):
    xb = x_ref[...] * dinv_ref[...]
    o_ref[...] = jnp.dot(xb, w_ref[...], preferred_element_type=jnp.float32)


def _m1_call(dinvc, x, W):
    return pl.pallas_call(
        _m1_body,
        grid=(N // _RB,),
        in_specs=[
            pl.BlockSpec((_RB, 1), lambda i: (i, 0)),
            pl.BlockSpec((_RB, F), lambda i: (i, 0)),
            pl.BlockSpec((F, F), lambda i: (0, 0)),
        ],
        out_specs=pl.BlockSpec((_RB, F), lambda i: (i, 0)),
        out_shape=jax.ShapeDtypeStruct((N, F), jnp.float32),
    )(dinvc, x, W)


def _m2_body(dinv_ref, p_ref, b_ref, w_ref, o_ref):
    agg = p_ref[0] + p_ref[1]
    h1 = jnp.maximum(agg * dinv_ref[...] + b_ref[...], 0.0)
    o_ref[...] = jnp.dot(h1 * dinv_ref[...], w_ref[...],
                         preferred_element_type=jnp.float32)


def _m2_call(dinvc, p, b1, W):
    return pl.pallas_call(
        _m2_body,
        grid=(N // _RB,),
        in_specs=[
            pl.BlockSpec((_RB, 1), lambda i: (i, 0)),
            pl.BlockSpec((NC, _RB, F), lambda i: (0, i, 0)),
            pl.BlockSpec((1, F), lambda i: (0, 0)),
            pl.BlockSpec((F, F), lambda i: (0, 0)),
        ],
        out_specs=pl.BlockSpec((_RB, F), lambda i: (i, 0)),
        out_shape=jax.ShapeDtypeStruct((N, F), jnp.float32),
    )(dinvc, p, b1, W)


def _fin_body(dinv_ref, p_ref, b_ref, o_ref):
    o_ref[...] = (p_ref[0] + p_ref[1]) * dinv_ref[...] + b_ref[...]


def _fin_call(dinvc, p, b2):
    return pl.pallas_call(
        _fin_body,
        grid=(N // _RB,),
        in_specs=[
            pl.BlockSpec((_RB, 1), lambda i: (i, 0)),
            pl.BlockSpec((NC, _RB, F), lambda i: (0, i, 0)),
            pl.BlockSpec((1, F), lambda i: (0, 0)),
        ],
        out_specs=pl.BlockSpec((_RB, F), lambda i: (i, 0)),
        out_shape=jax.ShapeDtypeStruct((N, F), jnp.float32),
    )(dinvc, p, b2)


def kernel(xs, edge_indexs, W1, b1, W2, b2):
    B = xs.shape[0]
    srcs = edge_indexs[:, 0, :].reshape(B, NW, NCHUNK, K)
    dsts = edge_indexs[:, 1, :].reshape(B, NW, NCHUNK, K)
    degp = _hist_kernel(dsts).reshape(2, 2, NPAD)
    dinv = _dinv_call(degp)
    dinvc = dinv[:, :N, None]
    zer = jnp.zeros((N, F), jnp.float32)
    b1r = b1.reshape(1, F)
    b2r = b2.reshape(1, F)
    outs = []
    for b in range(B):
        g1 = _m1_call(dinvc[b], xs[b], W1)
        p1 = _agg_kernel(g1, zer, srcs[b], dsts[b])
        g2 = _m2_call(dinvc[b], p1, b1r, W2)
        p2 = _agg_kernel(g2, zer, srcs[b], dsts[b])
        o = _fin_call(dinvc[b], p2, b2r)
        outs.append(o[None])
    return jnp.concatenate(outs, axis=0)
